# Initial kernel scaffold; baseline (speedup 1.0000x reference)
import jax
import jax.numpy as jnp
from jax.experimental import pallas as pl


def kernel(prop_embed, adap_embed, memMatrix, keyMatrix, x_proj_w, x_proj_b, w_gate, w_noise):
    raise NotImplementedError("write your pallas kernel here")


# speedup vs baseline: 2.5777x; 2.5777x over previous
import functools
import math

import jax
import jax.numpy as jnp
from jax.experimental import pallas as pl
from jax.experimental.pallas import tpu as pltpu

NEG_INF = float("-inf")
BIG_I32 = 2**30


def _body(nb, mb, rows, m_total, topk,
          prop_ref, xw_ref, xb_ref, wg_ref, key_ref, mem_ref,
          attw_ref, mlab_ref, memret_ref, lblret_ref,
          xq_s, topv_s, topi_s, wsm_s, att_s, accm_s, accl_s):
    i = pl.program_id(0)
    b = jnp.where(i < nb, i, i - nb)
    col0 = b * mb
    lane = jax.lax.broadcasted_iota(jnp.int32, (rows, 128), 1)

    @pl.when(i == 0)
    def _init():
        x = jnp.mean(prop_ref[...], axis=1).reshape(rows, 128)
        xq = jnp.tanh(
            jax.lax.dot_general(x, xw_ref[...], (((1,), (1,)), ((), ())),
                                preferred_element_type=jnp.float32)
            + xb_ref[...])
        xq_s[...] = xq
        topv_s[...] = jnp.full((rows, 128), NEG_INF, jnp.float32)
        topi_s[...] = jnp.zeros((rows, 128), jnp.int32)
        accm_s[...] = jnp.zeros_like(accm_s)
        accl_s[...] = jnp.zeros_like(accl_s)

    rowid = col0 + jax.lax.broadcasted_iota(jnp.int32, (mb, 128), 0)
    mblk = jnp.where(rowid < m_total, mem_ref[...], 0.0)
    colids = col0 + jax.lax.broadcasted_iota(jnp.int32, (rows, mb), 1)

    @pl.when(i < nb)
    def _pass0():
        att = jax.lax.dot_general(xq_s[...], key_ref[...],
                                  (((1,), (1,)), ((), ())),
                                  preferred_element_type=jnp.float32)
        att = jnp.where(colids < m_total, att, NEG_INF)
        att_s[...] = att
        logits = jax.lax.dot_general(mblk, wg_ref[...], (((1,), (0,)), ((), ())),
                                     preferred_element_type=jnp.float32)
        mlab_ref[...] = jax.nn.softmax(logits, axis=-1)

        def round_(_, carry):
            topv, topi = carry
            att = att_s[...]
            m = jnp.max(att, axis=1, keepdims=True)
            am = jnp.min(jnp.where(att == m, colids, BIG_I32), axis=1,
                         keepdims=True)
            att_s[...] = jnp.where(colids == am, NEG_INF, att)
            pos = jnp.sum(
                jnp.where(jnp.logical_and(topv >= m, lane < topk), 1, 0),
                axis=1, keepdims=True)
            rolv = jnp.concatenate([topv[:, :1], topv[:, :-1]], axis=1)
            roli = jnp.concatenate([topi[:, :1], topi[:, :-1]], axis=1)
            topv = jnp.where(lane < pos, topv,
                             jnp.where(lane == pos, m, rolv))
            topi = jnp.where(lane < pos, topi,
                             jnp.where(lane == pos, am, roli))
            topv = jnp.where(lane < topk, topv, NEG_INF)
            return topv, topi

        topv, topi = jax.lax.fori_loop(
            0, topk, round_, (topv_s[...], topi_s[...]))
        topv_s[...] = topv
        topi_s[...] = topi

        @pl.when(i == nb - 1)
        def _softmax_top():
            e = jnp.exp(topv - topv[:, :1])
            e = jnp.where(lane < topk, e, 0.0)
            wsm_s[...] = e / jnp.sum(e, axis=1, keepdims=True)

    @pl.when(i >= nb)
    def _pass1():
        topi = topi_s[...]
        wsm = wsm_s[...]
        attw = jnp.zeros((rows, mb), jnp.float32)
        for j in range(topk):
            attw = jnp.where(colids == topi[:, j:j + 1], wsm[:, j:j + 1],
                             attw)
        attw_ref[...] = attw
        accm_s[...] += jax.lax.dot_general(attw, mblk,
                                           (((1,), (0,)), ((), ())),
                                           preferred_element_type=jnp.float32)
        logits = jax.lax.dot_general(mblk, wg_ref[...], (((1,), (0,)), ((), ())),
                                     preferred_element_type=jnp.float32)
        mlab = jax.nn.softmax(logits, axis=-1)
        mlab = jnp.where(rowid[:, :8] < m_total, mlab, 0.0)
        accl_s[...] += jax.lax.dot_general(attw, mlab,
                                           (((1,), (0,)), ((), ())),
                                           preferred_element_type=jnp.float32)

        @pl.when(i == 2 * nb - 1)
        def _emit():
            memret_ref[...] = accm_s[...]
            lblret_ref[...] = accl_s[...]


def kernel(prop_embed, adap_embed, memMatrix, keyMatrix, x_proj_w, x_proj_b,
           w_gate, w_noise):
    B, T, N, D = prop_embed.shape
    M = memMatrix.shape[0]
    E = w_gate.shape[1]
    rows = B * N
    topk = int(math.log(M))
    mb = 2048
    nb = (M + mb - 1) // mb

    xb2 = x_proj_b.reshape(1, D)

    grid = (2 * nb,)
    body = functools.partial(_body, nb, mb, rows, M, topk)
    out_shapes = [
        jax.ShapeDtypeStruct((rows, M), jnp.float32),
        jax.ShapeDtypeStruct((M, E), jnp.float32),
        jax.ShapeDtypeStruct((rows, D), jnp.float32),
        jax.ShapeDtypeStruct((rows, E), jnp.float32),
    ]
    attw, mem_label, mem_ret, lbl_ret = pl.pallas_call(
        body,
        grid=grid,
        in_specs=[
            pl.BlockSpec((B, T, N, D), lambda i: (0, 0, 0, 0)),
            pl.BlockSpec((D, D), lambda i: (0, 0)),
            pl.BlockSpec((1, D), lambda i: (0, 0)),
            pl.BlockSpec((D, E), lambda i: (0, 0)),
            pl.BlockSpec((mb, D), lambda i: (jnp.minimum(i, nb - 1), 0)),
            pl.BlockSpec((mb, D), lambda i: (jnp.where(i < nb, i, i - nb), 0)),
        ],
        out_specs=[
            pl.BlockSpec((rows, mb), lambda i: (0, jnp.maximum(i - nb, 0))),
            pl.BlockSpec((mb, E), lambda i: (jnp.minimum(i, nb - 1), 0)),
            pl.BlockSpec((rows, D), lambda i: (0, 0)),
            pl.BlockSpec((rows, E), lambda i: (0, 0)),
        ],
        out_shape=out_shapes,
        scratch_shapes=[
            pltpu.VMEM((rows, 128), jnp.float32),
            pltpu.VMEM((rows, 128), jnp.float32),
            pltpu.VMEM((rows, 128), jnp.int32),
            pltpu.VMEM((rows, 128), jnp.float32),
            pltpu.VMEM((rows, mb), jnp.float32),
            pltpu.VMEM((rows, D), jnp.float32),
            pltpu.VMEM((rows, E), jnp.float32),
        ],
        compiler_params=pltpu.CompilerParams(
            dimension_semantics=("arbitrary",)),
    )(prop_embed, x_proj_w, xb2, w_gate, keyMatrix, memMatrix)

    att_weight = attw.reshape(B, N, M)
    mem_retrieved = mem_ret.reshape(B, N, D)
    label_retrieved = lbl_ret.reshape(B, N, E)
    return (label_retrieved, mem_retrieved, label_retrieved, mem_label,
            att_weight)
